# baseline (speedup 1.0000x reference)
import functools

import jax
import jax.numpy as jnp
from jax import lax
from jax.experimental import pallas as pl
from jax.experimental.pallas import tpu as pltpu
from jax.experimental.pallas import tpu_sc as plsc

N = 10000
E = 320000
D = 128
H = 4
DH = 32

NC = 2
NS = 16
NW = NC * NS
EPW = E // NW
C = 80
NCHUNK = EPW // C

PK = 2512
RNG = 624
NROUND = 17
NPAD2 = RNG * NROUND

_f32 = jnp.float32


def _k0_body(h_ref, xp_ref, we1r_ref, we1c_ref, be1_ref, wars_ref, wacs_ref,
             sxs_ref, ur_ref, uc_ref, pr_ref, pc_ref):
    hb = h_ref[...]
    ur_ref[...] = hb @ we1r_ref[...] + be1_ref[...]
    uc_ref[...] = hb @ we1c_ref[...]
    xs = xp_ref[...] @ sxs_ref[...]
    pr_ref[...] = hb @ wars_ref[...] + xs
    pc_ref[...] = hb @ wacs_ref[...] + xs


def _run_k0(h, xpad, We1r, We1c, be1, WarS, WacS, SxS):
    bn = 1000
    grid = N // bn
    full = lambda shape: pl.BlockSpec(shape, lambda i: (0, 0))
    return pl.pallas_call(
        _k0_body,
        grid=(grid,),
        in_specs=[
            pl.BlockSpec((bn, D), lambda i: (i, 0)),
            pl.BlockSpec((bn, 8), lambda i: (i, 0)),
            full((D, D)), full((D, D)), full((1, D)),
            full((D, 16)), full((D, 16)), full((8, 16)),
        ],
        out_specs=[
            pl.BlockSpec((bn, D), lambda i: (i, 0)),
            pl.BlockSpec((bn, D), lambda i: (i, 0)),
            pl.BlockSpec((bn, 16), lambda i: (i, 0)),
            pl.BlockSpec((bn, 16), lambda i: (i, 0)),
        ],
        out_shape=[
            jax.ShapeDtypeStruct((N, D), _f32),
            jax.ShapeDtypeStruct((N, D), _f32),
            jax.ShapeDtypeStruct((N, 16), _f32),
            jax.ShapeDtypeStruct((N, 16), _f32),
        ],
    )(h, xpad, We1r, We1c, be1, WarS, WacS, SxS)


def _lane():
    return lax.broadcasted_iota(jnp.int32, (16,), 0)


def _silu(v):
    return v * (1.0 / (1.0 + jnp.exp(-v)))


_GDN = lax.GatherDimensionNumbers(
    offset_dims=(), collapsed_slice_dims=(0,), start_index_map=(0,))


def _splat(v, k):
    idx = jnp.full((16, 1), k, jnp.int32)
    return lax.gather(v, idx, _GDN, (1,),
                      mode=lax.GatherScatterMode.PROMISE_IN_BOUNDS)


def _gath(v, idx16):
    return lax.gather(v, idx16.reshape(16, 1), _GDN, (1,),
                      mode=lax.GatherScatterMode.PROMISE_IN_BOUNDS)


def _make_pass_a():
    mesh = plsc.VectorSubcoreMesh(core_axis_name="c", subcore_axis_name="s")

    @functools.partial(
        pl.kernel,
        mesh=mesh,
        compiler_params=pltpu.CompilerParams(use_tc_tiling_on_sc=False),
        out_type=[
            jax.ShapeDtypeStruct((E, D), _f32),
            jax.ShapeDtypeStruct((E, 16), _f32),
            jax.ShapeDtypeStruct((NW * PK, 16), _f32),
        ],
        scratch_types=[
            pltpu.VMEM((C,), jnp.int32),
            pltpu.VMEM((C,), jnp.int32),
            pltpu.VMEM((C * 4 + 16,), _f32),
            pltpu.VMEM((C, D), _f32),
            pltpu.VMEM((C, D), _f32),
            pltpu.VMEM((C, 16), _f32),
            pltpu.VMEM((C, 16), _f32),
            pltpu.VMEM((C, D), _f32),
            pltpu.VMEM((C, 16), _f32),
            pltpu.VMEM((PK, 16), _f32),
            pltpu.VMEM((D,), _f32),
            pltpu.VMEM((4, D), _f32),
            pltpu.SemaphoreType.DMA,
        ],
    )
    def kern(row_hbm, col_hbm, ea_hbm, ur_hbm, uc_hbm, pr_hbm, pc_hbm,
             wsq_hbm, wea_hbm, m1_hbm, aux_hbm, ssum_hbm,
             rowv, colv, eav, gru, gcu, gpr, gpc, m1buf, auxbuf,
             acc, wsqv, weav, sem):
        cid = lax.axis_index("c")
        sid = lax.axis_index("s")
        wid = sid * NC + cid
        lane = _lane()
        z16 = jnp.zeros((16,), _f32)

        pltpu.sync_copy(wsq_hbm, wsqv)
        pltpu.sync_copy(wea_hbm, weav)

        def _z(i, _):
            acc[i, :] = z16
            return 0
        lax.fori_loop(0, PK, _z, 0)

        def chunk(i, _):
            base = wid * EPW + i * C
            pltpu.sync_copy(row_hbm.at[pl.ds(base, C)], rowv)
            pltpu.sync_copy(col_hbm.at[pl.ds(base, C)], colv)
            pltpu.sync_copy(ea_hbm.at[pl.ds(base * 4, C * 4)],
                            eav.at[pl.ds(0, C * 4)])
            c1 = pltpu.async_copy(ur_hbm.at[rowv], gru, sem)
            c2 = pltpu.async_copy(uc_hbm.at[colv], gcu, sem)
            c3 = pltpu.async_copy(pr_hbm.at[rowv], gpr, sem)
            c4 = pltpu.async_copy(pc_hbm.at[colv], gpc, sem)
            c1.wait(); c2.wait(); c3.wait(); c4.wait()

            def edge(e, _):
                pr16 = gpr[e, :]
                pc16 = gpc[e, :]
                d16 = pr16 - pc16
                s16 = pr16 + pc16
                lg = jnp.where(s16 >= 0, s16, 0.2 * s16)
                p16 = jnp.where((lane >= 3) & (lane < 7), jnp.exp(lg), 0.0)
                dd = jnp.where(lane < 3, d16 * d16, 0.0)
                sq = _splat(dd, 0) + _splat(dd, 1) + _splat(dd, 2)
                auxbuf[e, :] = jnp.where(lane < 3, d16, p16)
                n = rowv[pl.ds(e, 16)][0]
                sh = (n & 3) * 4
                pm = jnp.where((lane >= sh) & (lane < sh + 4),
                               _gath(p16, jnp.clip(lane - sh + 3, 0, 15)),
                               0.0)
                pr_ = n >> 2
                acc[pr_, :] = acc[pr_, :] + pm
                ev = eav[pl.ds(4 * e, 16)]
                e0 = _splat(ev, 0)
                e1 = _splat(ev, 1)
                e2 = _splat(ev, 2)
                e3 = _splat(ev, 3)
                for j in range(D // 16):
                    s = pl.ds(16 * j, 16)
                    v = (gru[e, s] + gcu[e, s] + sq * wsqv[s]
                         + e0 * weav[0, s] + e1 * weav[1, s]
                         + e2 * weav[2, s] + e3 * weav[3, s])
                    m1buf[e, s] = _silu(v)
                return 0
            lax.fori_loop(0, C, edge, 0)

            pltpu.sync_copy(m1buf, m1_hbm.at[pl.ds(base, C)])
            pltpu.sync_copy(auxbuf, aux_hbm.at[pl.ds(base, C)])
            return 0
        lax.fori_loop(0, NCHUNK, chunk, 0)

        pltpu.sync_copy(acc, ssum_hbm.at[pl.ds(wid * PK, PK)])

    return kern


def _red_body(in_ref, out_ref):
    t = pl.program_id(0)

    @pl.when(t == 0)
    def _():
        out_ref[...] = jnp.zeros_like(out_ref)
    out_ref[...] += in_ref[...]


def _run_reduce_pk(x):
    return pl.pallas_call(
        _red_body,
        grid=(NW,),
        in_specs=[pl.BlockSpec((PK, 16), lambda t: (t, 0))],
        out_specs=pl.BlockSpec((PK, 16), lambda t: (0, 0)),
        out_shape=jax.ShapeDtypeStruct((PK, 16), _f32),
    )(x)


def _red3_body(in_ref, out_ref):
    t = pl.program_id(1)

    @pl.when(t == 0)
    def _():
        out_ref[...] = jnp.zeros_like(out_ref)
    out_ref[...] += in_ref[...]


def _run_reduce_hagg(x):
    return pl.pallas_call(
        _red3_body,
        grid=(NROUND, NW),
        in_specs=[pl.BlockSpec((RNG, D), lambda r, t: (t * NROUND + r, 0))],
        out_specs=pl.BlockSpec((RNG, D), lambda r, t: (r, 0)),
        out_shape=jax.ShapeDtypeStruct((NPAD2, D), _f32),
    )(x)


def _kb_body(m1_ref, we2_ref, be2_ref, wm_ref, bm_ref, wx2_ref,
             msg_ref, cm_ref):
    m1 = m1_ref[...]
    msg_ref[...] = m1 @ we2_ref[...] + be2_ref[...]
    t = m1 @ wm_ref[...] + bm_ref[...]
    cm_ref[...] = _silu(t) @ wx2_ref[...]


def _run_kb(m1, We2, be2, Wm, bm, Wx2):
    be = 512
    grid = E // be
    full = lambda shape: pl.BlockSpec(shape, lambda i: (0, 0))
    return pl.pallas_call(
        _kb_body,
        grid=(grid,),
        in_specs=[
            pl.BlockSpec((be, D), lambda i: (i, 0)),
            full((D, D)), full((1, D)), full((D, D)), full((1, D)),
            full((D, 1)),
        ],
        out_specs=[
            pl.BlockSpec((be, D), lambda i: (i, 0)),
            pl.BlockSpec((be, 1), lambda i: (i, 0)),
        ],
        out_shape=[
            jax.ShapeDtypeStruct((E, D), _f32),
            jax.ShapeDtypeStruct((E, 1), _f32),
        ],
    )(m1, We2, be2, Wm, bm, Wx2)


def _make_pass_c1():
    mesh = plsc.VectorSubcoreMesh(core_axis_name="c", subcore_axis_name="s")

    @functools.partial(
        pl.kernel,
        mesh=mesh,
        compiler_params=pltpu.CompilerParams(use_tc_tiling_on_sc=False),
        out_type=[
            jax.ShapeDtypeStruct((NW * NPAD2, D), _f32),
            jax.ShapeDtypeStruct((E, D), _f32),
        ],
        scratch_types=[
            pltpu.VMEM((C,), jnp.int32),
            pltpu.VMEM((C,), jnp.int32),
            pltpu.VMEM((C,), jnp.int32),
            pltpu.VMEM((C, D), _f32),
            pltpu.VMEM((C, 16), _f32),
            pltpu.VMEM((C, 16), _f32),
            pltpu.VMEM((RNG, D), _f32),
            pltpu.SemaphoreType.DMA,
        ],
    )
    def kern(row_hbm, col_hbm, aux_hbm, msg_hbm, spk_hbm,
             hagg_hbm, mw_hbm,
             rowv, rv4, colv, msgv, auxv, g0, acc, sem):
        cid = lax.axis_index("c")
        sid = lax.axis_index("s")
        wid = sid * NC + cid
        lane = _lane()
        z16 = jnp.zeros((16,), _f32)

        def chunk(i, _):
            base = wid * EPW + i * C
            pltpu.sync_copy(row_hbm.at[pl.ds(base, C)], rowv)
            pltpu.sync_copy(aux_hbm.at[pl.ds(base, C)], auxv)
            pltpu.sync_copy(msg_hbm.at[pl.ds(base, C)], msgv)

            def mkidx(k, _):
                s = pl.ds(16 * k, 16)
                rv4[s] = lax.shift_right_logical(rowv[s], 2)
                return 0
            lax.fori_loop(0, C // 16, mkidx, 0)
            pltpu.async_copy(spk_hbm.at[rv4], g0, sem).wait()

            def edge(e, _):
                aux16 = auxv[e, :]
                n = rowv[pl.ds(e, 16)][0]
                sh = (n & 3) * 4
                S16 = _gath(g0[e, :], jnp.clip(lane - 3 + sh, 0, 15))
                w16 = aux16 / (S16 + 1e-16)
                for hh in range(H):
                    wh = _splat(w16, 3 + hh)
                    for j2 in range(2):
                        s = pl.ds((2 * hh + j2) * 16, 16)
                        msgv[e, s] = msgv[e, s] * wh
                return 0
            lax.fori_loop(0, C, edge, 0)
            pltpu.sync_copy(msgv, mw_hbm.at[pl.ds(base, C)])
            return 0
        lax.fori_loop(0, NCHUNK, chunk, 0)

        def rnd(r, _):
            lo = r * RNG

            def _z(i, _):
                for j in range(D // 16):
                    acc[i, pl.ds(16 * j, 16)] = z16
                return 0
            lax.fori_loop(0, RNG, _z, 0)

            def chunk2(i, _):
                base = wid * EPW + i * C
                pltpu.sync_copy(col_hbm.at[pl.ds(base, C)], colv)
                pltpu.sync_copy(mw_hbm.at[pl.ds(base, C)], msgv)

                def edge(e, _):
                    n = colv[pl.ds(e, 16)][0]
                    inr = (n >= lo) & (n < lo + RNG)

                    @pl.when(inr)
                    def _():
                        rr = n - lo
                        for j in range(D // 16):
                            s = pl.ds(16 * j, 16)
                            acc[rr, s] = acc[rr, s] + msgv[e, s]
                    return 0
                lax.fori_loop(0, C, edge, 0)
                return 0
            lax.fori_loop(0, NCHUNK, chunk2, 0)

            pltpu.sync_copy(acc, hagg_hbm.at[pl.ds(wid * NPAD2 + lo, RNG)])
            return 0
        lax.fori_loop(0, NROUND, rnd, 0)

    return kern


def _make_pass_c2():
    mesh = plsc.VectorSubcoreMesh(core_axis_name="c", subcore_axis_name="s")

    @functools.partial(
        pl.kernel,
        mesh=mesh,
        compiler_params=pltpu.CompilerParams(use_tc_tiling_on_sc=False),
        out_type=[
            jax.ShapeDtypeStruct((NW * PK, 16), _f32),
        ],
        scratch_types=[
            pltpu.VMEM((C,), jnp.int32),
            pltpu.VMEM((C, 16), _f32),
            pltpu.VMEM((C + 16,), _f32),
            pltpu.VMEM((PK, 16), _f32),
            pltpu.SemaphoreType.DMA,
        ],
    )
    def kern(col_hbm, aux_hbm, cm_hbm, naux_hbm,
             colv, auxv, cmv, acc, sem):
        cid = lax.axis_index("c")
        sid = lax.axis_index("s")
        wid = sid * NC + cid
        lane = _lane()
        z16 = jnp.zeros((16,), _f32)

        def _z(i, _):
            acc[i, :] = z16
            return 0
        lax.fori_loop(0, PK, _z, 0)

        def chunk(i, _):
            base = wid * EPW + i * C
            pltpu.sync_copy(col_hbm.at[pl.ds(base, C)], colv)
            pltpu.sync_copy(aux_hbm.at[pl.ds(base, C)], auxv)
            pltpu.sync_copy(cm_hbm.at[pl.ds(base, C)], cmv.at[pl.ds(0, C)])

            def edge(e, _):
                aux16 = auxv[e, :]
                cm = _splat(cmv[pl.ds(e, 16)], 0)
                rc = jnp.where(
                    lane < 3, aux16 * cm,
                    jnp.where(lane == 3, jnp.full((16,), 1.0, _f32), 0.0))
                n = colv[pl.ds(e, 16)][0]
                sh = (n & 3) * 4
                rot = jnp.where((lane >= sh) & (lane < sh + 4),
                                _gath(rc, jnp.clip(lane - sh, 0, 15)), 0.0)
                pr_ = n >> 2
                acc[pr_, :] = acc[pr_, :] + rot
                return 0
            lax.fori_loop(0, C, edge, 0)
            return 0
        lax.fori_loop(0, NCHUNK, chunk, 0)

        pltpu.sync_copy(acc, naux_hbm.at[pl.ds(wid * PK, PK)])

    return kern


def _kd_body(h_ref, xp_ref, hagg_ref, na_ref,
             wh1a_ref, wh1b_ref, bh1_ref, wh2_ref, bh2_ref,
             sn_ref, sc_ref, hout_ref, xout_ref):
    h = h_ref[...]
    u = h @ wh1a_ref[...] + hagg_ref[...] @ wh1b_ref[...] + bh1_ref[...]
    hout_ref[...] = h + _silu(u) @ wh2_ref[...] + bh2_ref[...]
    na = na_ref[...]
    num8 = na @ sn_ref[...]
    cnt8 = na @ sc_ref[...]
    xout_ref[...] = xp_ref[...] + num8 / jnp.maximum(cnt8, 1.0)


def _run_kd(h, xpad, hagg, naux4, Wh1a, Wh1b, bh1, Wh2, bh2, Sn4, Sc4):
    bn = 1000
    grid = N // bn
    full = lambda shape: pl.BlockSpec(shape, lambda i: (0, 0))
    row = lambda w: pl.BlockSpec((bn, w), lambda i: (i, 0))
    return pl.pallas_call(
        _kd_body,
        grid=(grid,),
        in_specs=[
            row(D), row(8), row(D), row(4),
            full((D, D)), full((D, D)), full((1, D)), full((D, D)),
            full((1, D)), full((4, 8)), full((4, 8)),
        ],
        out_specs=[row(D), row(8)],
        out_shape=[
            jax.ShapeDtypeStruct((N, D), _f32),
            jax.ShapeDtypeStruct((N, 8), _f32),
        ],
    )(h, xpad, hagg, naux4, Wh1a, Wh1b, bh1, Wh2, bh2, Sn4, Sc4)


_PASS_A = _make_pass_a()
_PASS_C1 = _make_pass_c1()
_PASS_C2 = _make_pass_c2()


def kernel(h, x_coord, edge_index, edge_attr, W_att, We1, be1, We2, be2,
           Wx1, bx1, Wx2, Wh1, bh1, Wh2, bh2):
    row = edge_index[0].astype(jnp.int32)
    col = edge_index[1].astype(jnp.int32)
    xpad = jnp.pad(x_coord, ((0, 0), (0, 5)))

    We1r = We1[:D]
    We1c = We1[D:2 * D]
    wsq = We1[2 * D]
    Wea = We1[2 * D + 1:]
    Sa = jnp.zeros((4, 16), _f32).at[jnp.arange(4), jnp.arange(3, 7)].set(1.0)
    Sx = jnp.zeros((8, 16), _f32).at[jnp.arange(3), jnp.arange(3)].set(1.0)
    WarS = W_att[:D] @ Sa
    WacS = W_att[D:] @ Sa
    Pm = (jnp.tile(jnp.eye(DH, dtype=_f32), (H, 1)) / H)
    PW = Pm @ Wx1
    Wm = We2 @ PW
    bm = (be2 @ PW + bx1)[None, :]
    Sn4 = jnp.zeros((4, 8), _f32).at[jnp.arange(3), jnp.arange(3)].set(1.0)
    Sc4 = jnp.zeros((4, 8), _f32).at[3, :].set(1.0)

    Ur, Uc, Pr, Pc = _run_k0(h, xpad, We1r, We1c, be1[None, :], WarS, WacS, Sx)

    m1, aux, sspacks = _PASS_A(row, col, edge_attr.reshape(-1),
                               Ur, Uc, Pr, Pc, wsq, Wea)
    sspk = _run_reduce_pk(sspacks)

    msgs, cm = _run_kb(m1, We2, be2[None, :], Wm, bm, Wx2)

    haggp, _ = _PASS_C1(row, col, aux, msgs, sspk)
    hagg = _run_reduce_hagg(haggp)[:N]

    napacks, = _PASS_C2(col, aux, cm.reshape(-1))
    naux4 = _run_reduce_pk(napacks).reshape(PK * 4, 4)[:N]

    h_out, x_out8 = _run_kd(h, xpad, hagg, naux4,
                            Wh1[:D], Wh1[D:], bh1[None, :], Wh2, bh2[None, :],
                            Sn4, Sc4)
    return (h_out, x_out8[:, :3])
